# Initial kernel scaffold; baseline (speedup 1.0000x reference)
import jax
import jax.numpy as jnp
from jax.experimental import pallas as pl


def kernel(x, edge_index, edge_weight, Wl0, Wr0, att0, b0, g0, be0, al0, Wl1, Wr1, att1, b1, g1, be1, al1):
    raise NotImplementedError("write your pallas kernel here")


# speedup vs baseline: 7.0922x; 7.0922x over previous
import functools

import jax
import jax.numpy as jnp
from jax import lax
from jax.experimental import pallas as pl
from jax.experimental.pallas import tpu as pltpu
from jax.experimental.pallas import tpu_sc as plsc

N = 10000
E = 320000
D = 128
DH = 64
H = 4
C = 32

NC = 2
NS = 16
EPW = E // (NC * NS)
K = 80
NCHUNK = EPW // K
NPS = 632
NPAD = NPS * NS
DP = 16


def _iota16():
    return lax.iota(jnp.int32, 16)


def _full16(v):
    return jnp.full((16,), v, jnp.int32)


def _pass1_body(xll_hbm, xlh_hbm, xr_hbm, src_hbm, dst_hbm, att_hbm, zden_hbm,
                ex_hbm, den_hbm,
                att_v, srcv, dstv, xlv_lo, xlv_hi, xrv, exrow, exout, zbuf,
                den_sp, sem1, sem2, sem3):
    c = lax.axis_index("c")
    s = lax.axis_index("s")
    base_w = (c * NS + s) * EPW

    pltpu.sync_copy(zden_hbm.at[pl.ds(s * NPS, NPS)], zbuf)
    pltpu.sync_copy(zbuf, den_sp.at[pl.ds(s * NPS, NPS)])
    pltpu.sync_copy(att_hbm, att_v)

    zero16 = jnp.zeros((16,), jnp.float32)
    for g in range(K // 16):
        eidx0 = _iota16() + g * 16
        for j in range(H, DP):
            plsc.store_scatter(exrow, [eidx0, _full16(j)], zero16)

    plsc.subcore_barrier()

    def chunk_body(i, carry):
        base = base_w + i * K
        pltpu.sync_copy(src_hbm.at[pl.ds(base, K)], srcv)
        pltpu.sync_copy(dst_hbm.at[pl.ds(base, K)], dstv)
        cp1 = pltpu.async_copy(xll_hbm.at[srcv], xlv_lo, sem1)
        cp2 = pltpu.async_copy(xlh_hbm.at[srcv], xlv_hi, sem2)
        cp3 = pltpu.async_copy(xr_hbm.at[dstv], xrv, sem3)
        cp1.wait()
        cp2.wait()
        cp3.wait()

        att_vecs = [att_v[pl.ds(j * 16, 16)] for j in range(D // 16)]

        def group_body(g, carry2):
            eidx = _iota16() + g * 16
            accs = [jnp.zeros((16,), jnp.float32) for _ in range(H)]
            for ch in range(D):
                if ch < DH:
                    a = plsc.load_gather(xlv_lo, [eidx, _full16(ch)])
                else:
                    a = plsc.load_gather(xlv_hi, [eidx, _full16(ch - DH)])
                b = plsc.load_gather(xrv, [eidx, _full16(ch)])
                z = a + b
                z = jnp.maximum(z, 0.2 * z)
                accs[ch >> 5] = accs[ch >> 5] + z * att_vecs[ch >> 4][ch & 15]
            for h in range(H):
                exv = jnp.exp(accs[h])
                plsc.store_scatter(exrow, [eidx, _full16(h)], exv)
                plsc.store_scatter(exout, [eidx, _full16(h)], exv)
            return carry2

        lax.fori_loop(0, K // 16, group_body, 0)
        pltpu.sync_copy(exout, ex_hbm.at[pl.ds(base, K)])
        pltpu.sync_copy(exrow, den_sp.at[dstv], add=True)
        return carry

    lax.fori_loop(0, NCHUNK, chunk_body, 0)
    plsc.subcore_barrier()
    pltpu.sync_copy(den_sp.at[pl.ds(s * NPS, NPS)], zbuf)
    pltpu.sync_copy(zbuf, den_hbm.at[c, pl.ds(s * NPS, NPS)])


@functools.cache
def _get_pass1():
    return pl.kernel(
        _pass1_body,
        out_type=(
            jax.ShapeDtypeStruct((E, H), jnp.float32),
            jax.ShapeDtypeStruct((NC, NPAD, DP), jnp.float32),
        ),
        mesh=plsc.VectorSubcoreMesh(core_axis_name="c", subcore_axis_name="s",
                                    num_cores=NC, num_subcores=NS),
        compiler_params=pltpu.CompilerParams(needs_layout_passes=False,
                                             use_tc_tiling_on_sc=False),
        scratch_types=(
            pltpu.VMEM((D,), jnp.float32),
            pltpu.VMEM((K,), jnp.int32),
            pltpu.VMEM((K,), jnp.int32),
            pltpu.VMEM((K, DH), jnp.float32),
            pltpu.VMEM((K, DH), jnp.float32),
            pltpu.VMEM((K, D), jnp.float32),
            pltpu.VMEM((K, DP), jnp.float32),
            pltpu.VMEM((K, H), jnp.float32),
            pltpu.VMEM((NPS, DP), jnp.float32),
            pltpu.VMEM_SHARED((NPAD, DP), jnp.float32),
            pltpu.SemaphoreType.DMA,
            pltpu.SemaphoreType.DMA,
            pltpu.SemaphoreType.DMA,
        ),
    )


def _make_pass2_body(h0):
    def _pass2_body(xl_hbm, src_hbm, dst_hbm, w_hbm, ex_hbm, den0_hbm,
                    den1_hbm, zout_hbm,
                    out_hbm,
                    srcv, dstv, wv, exr, den0r, den1r, xlv, msgv, zbuf,
                    out_sp, sem1, sem2, sem3):
        c = lax.axis_index("c")
        s = lax.axis_index("s")
        base_w = (c * NS + s) * EPW

        pltpu.sync_copy(zout_hbm.at[pl.ds(s * NPS, NPS)], zbuf)
        pltpu.sync_copy(zbuf, out_sp.at[pl.ds(s * NPS, NPS)])
        plsc.subcore_barrier()

        def chunk_body(i, carry):
            base = base_w + i * K
            pltpu.sync_copy(src_hbm.at[pl.ds(base, K)], srcv)
            pltpu.sync_copy(dst_hbm.at[pl.ds(base, K)], dstv)
            pltpu.sync_copy(w_hbm.at[pl.ds(base, K)], wv)
            pltpu.sync_copy(ex_hbm.at[pl.ds(base, K)], exr)
            cp1 = pltpu.async_copy(xl_hbm.at[srcv], xlv, sem1)
            cp2 = pltpu.async_copy(den0_hbm.at[dstv], den0r, sem2)
            cp3 = pltpu.async_copy(den1_hbm.at[dstv], den1r, sem3)
            cp1.wait()
            cp2.wait()
            cp3.wait()

            def group_body(g, carry2):
                eidx = _iota16() + g * 16
                wvg = wv[pl.ds(g * 16, 16)]
                alphas = []
                for h in range(2):
                    hh = _full16(h0 + h)
                    e_h = plsc.load_gather(exr, [eidx, hh])
                    d_h = (plsc.load_gather(den0r, [eidx, hh])
                           + plsc.load_gather(den1r, [eidx, hh]))
                    alphas.append(e_h * wvg / (d_h + 1e-16))
                for ch in range(DH):
                    cc = _full16(ch)
                    col = plsc.load_gather(xlv, [eidx, cc])
                    plsc.store_scatter(msgv, [eidx, cc],
                                       col * alphas[ch >> 5])
                return carry2

            lax.fori_loop(0, K // 16, group_body, 0)
            pltpu.sync_copy(msgv, out_sp.at[dstv], add=True)
            return carry

        lax.fori_loop(0, NCHUNK, chunk_body, 0)
        plsc.subcore_barrier()
        pltpu.sync_copy(out_sp.at[pl.ds(s * NPS, NPS)], zbuf)
        pltpu.sync_copy(zbuf, out_hbm.at[c, pl.ds(s * NPS, NPS)])

    return _pass2_body


@functools.cache
def _get_pass2(h0):
    return pl.kernel(
        _make_pass2_body(h0),
        out_type=jax.ShapeDtypeStruct((NC, NPAD, DH), jnp.float32),
        mesh=plsc.VectorSubcoreMesh(core_axis_name="c", subcore_axis_name="s",
                                    num_cores=NC, num_subcores=NS),
        compiler_params=pltpu.CompilerParams(needs_layout_passes=False,
                                             use_tc_tiling_on_sc=False),
        scratch_types=(
            pltpu.VMEM((K,), jnp.int32),
            pltpu.VMEM((K,), jnp.int32),
            pltpu.VMEM((K,), jnp.float32),
            pltpu.VMEM((K, H), jnp.float32),
            pltpu.VMEM((K, DP), jnp.float32),
            pltpu.VMEM((K, DP), jnp.float32),
            pltpu.VMEM((K, DH), jnp.float32),
            pltpu.VMEM((K, DH), jnp.float32),
            pltpu.VMEM((NPS, DH), jnp.float32),
            pltpu.VMEM_SHARED((NPAD, DH), jnp.float32),
            pltpu.SemaphoreType.DMA,
            pltpu.SemaphoreType.DMA,
            pltpu.SemaphoreType.DMA,
        ),
    )


def _norm_proj(h, g, be, al, wl, wr):
    mean = jnp.mean(h, axis=0, keepdims=True)
    xc = h - al * mean
    var = jnp.mean(xc * xc, axis=0, keepdims=True)
    hn = g * xc / jnp.sqrt(var + 1e-5) + be
    ha = jnp.maximum(hn, 0.01 * hn)
    xl = jnp.dot(ha, wl, preferred_element_type=jnp.float32)
    xr = jnp.dot(ha, wr, preferred_element_type=jnp.float32)
    return xl, xr


def _tc1_body(h_ref, g_ref, be_ref, al_ref, wl_ref, wr_ref,
              xll_ref, xlh_ref, xr_ref):
    xl, xr = _norm_proj(h_ref[:, :], g_ref[:], be_ref[:], al_ref[:],
                        wl_ref[:, :], wr_ref[:, :])
    xll_ref[:, :] = xl[:, :DH]
    xlh_ref[:, :] = xl[:, DH:]
    xr_ref[:, :] = xr


_tc1 = pl.pallas_call(
    _tc1_body,
    out_shape=(
        jax.ShapeDtypeStruct((N, DH), jnp.float32),
        jax.ShapeDtypeStruct((N, DH), jnp.float32),
        jax.ShapeDtypeStruct((N, D), jnp.float32),
    ),
)


def _combine(plo_ref, phi_ref, b_ref, res_ref):
    lo = plo_ref[0] + plo_ref[1]
    hi = phi_ref[0] + phi_ref[1]
    p = jnp.concatenate([lo, hi], axis=1)[:N]
    return p + b_ref[:] + res_ref[:, :]


def _tc2_body(plo_ref, phi_ref, b_ref, res_ref, g_ref, be_ref, al_ref,
              wl_ref, wr_ref, h1_ref, xll_ref, xlh_ref, xr_ref):
    h1 = _combine(plo_ref, phi_ref, b_ref, res_ref)
    h1_ref[:, :] = h1
    xl, xr = _norm_proj(h1, g_ref[:], be_ref[:], al_ref[:],
                        wl_ref[:, :], wr_ref[:, :])
    xll_ref[:, :] = xl[:, :DH]
    xlh_ref[:, :] = xl[:, DH:]
    xr_ref[:, :] = xr


_tc2 = pl.pallas_call(
    _tc2_body,
    out_shape=(
        jax.ShapeDtypeStruct((N, D), jnp.float32),
        jax.ShapeDtypeStruct((N, DH), jnp.float32),
        jax.ShapeDtypeStruct((N, DH), jnp.float32),
        jax.ShapeDtypeStruct((N, D), jnp.float32),
    ),
)


def _tc3_body(plo_ref, phi_ref, b_ref, res_ref, out_ref):
    out_ref[:, :] = _combine(plo_ref, phi_ref, b_ref, res_ref)


_tc3 = pl.pallas_call(
    _tc3_body,
    out_shape=jax.ShapeDtypeStruct((N, D), jnp.float32),
)


@jax.jit
def _stage1(x, src, dst, edge_weight, Wl0, Wr0, att0f, g0, be0, al0,
            zden, zout):
    xll0, xlh0, xr0 = _tc1(x, g0, be0, al0, Wl0, Wr0)
    ex0, den0 = _get_pass1()(xll0, xlh0, xr0, src, dst, att0f, zden)
    p_lo0 = _get_pass2(0)(xll0, src, dst, edge_weight, ex0, den0[0], den0[1],
                          zout)
    return xlh0, ex0, den0, p_lo0


@jax.jit
def _stage2(xlh0, src, dst, edge_weight, ex0, den0, zout):
    return _get_pass2(2)(xlh0, src, dst, edge_weight, ex0, den0[0], den0[1],
                         zout)


@jax.jit
def _stage3(p_lo0, p_hi0, x, src, dst, edge_weight, Wl1, Wr1, att1f, b0,
            g1, be1, al1, zden, zout):
    h1, xll1, xlh1, xr1 = _tc2(p_lo0, p_hi0, b0, x, g1, be1, al1, Wl1, Wr1)
    ex1, den1 = _get_pass1()(xll1, xlh1, xr1, src, dst, att1f, zden)
    p_lo1 = _get_pass2(0)(xll1, src, dst, edge_weight, ex1, den1[0], den1[1],
                          zout)
    return h1, xlh1, ex1, den1, p_lo1


@jax.jit
def _stage4(p_lo1, h1, xlh1, src, dst, edge_weight, ex1, den1, b1, zout):
    p_hi1 = _get_pass2(2)(xlh1, src, dst, edge_weight, ex1, den1[0], den1[1],
                          zout)
    return _tc3(p_lo1, p_hi1, b1, h1)


def kernel(x, edge_index, edge_weight, Wl0, Wr0, att0, b0, g0, be0, al0,
           Wl1, Wr1, att1, b1, g1, be1, al1):
    src = edge_index[0]
    dst = edge_index[1]
    zden = jnp.zeros((NPAD, DP), jnp.float32)
    zout = jnp.zeros((NPAD, DH), jnp.float32)
    xlh0, ex0, den0, p_lo0 = _stage1(x, src, dst, edge_weight, Wl0, Wr0,
                                     att0.reshape(-1), g0, be0, al0,
                                     zden, zout)
    p_hi0 = _stage2(xlh0, src, dst, edge_weight, ex0, den0, zout)
    h1, xlh1, ex1, den1, p_lo1 = _stage3(p_lo0, p_hi0, x, src, dst,
                                         edge_weight, Wl1, Wr1,
                                         att1.reshape(-1), b0, g1, be1, al1,
                                         zden, zout)
    return _stage4(p_lo1, h1, xlh1, src, dst, edge_weight, ex1, den1, b1,
                   zout)
